# baseline (speedup 1.0000x reference)
import functools

import jax
import jax.numpy as jnp
from jax import lax
from jax.experimental import pallas as pl
from jax.experimental.pallas import tpu as pltpu
from jax.experimental.pallas import tpu_sc as plsc

NC = 2
NS = 16
L = 16

N = 10000
D = 128
DH = D // NC
CHUNK = 128
NQ = 4
RA = 624
TAIL = N - NS * RA


def _sc_aggregate(x2, src2d, dst2d, w2d, n_chunks):
    mesh = plsc.VectorSubcoreMesh(core_axis_name="c", subcore_axis_name="s")

    @functools.partial(
        pl.kernel,
        out_type=jax.ShapeDtypeStruct((NC, N, DH), jnp.float32),
        mesh=mesh,
        compiler_params=pltpu.CompilerParams(use_tc_tiling_on_sc=False),
        scratch_types=[
            pltpu.VMEM_SHARED((N, DH), jnp.float32),
            pltpu.VMEM_SHARED((N, DH), jnp.float32),
            pltpu.VMEM((n_chunks // NQ, CHUNK), jnp.int32),
            pltpu.VMEM((n_chunks // NQ, CHUNK), jnp.int32),
            pltpu.VMEM((n_chunks // NQ, CHUNK), jnp.float32),
            pltpu.VMEM((CHUNK, DH), jnp.float32),
            pltpu.VMEM((CHUNK, DH), jnp.float32),
            pltpu.SemaphoreType.DMA,
            pltpu.SemaphoreType.DMA,
            pltpu.SemaphoreType.DMA,
            pltpu.SemaphoreType.DMA,
        ],
    )
    def agg(x2_hbm, src_hbm, dst_hbm, w_hbm, out_hbm,
            xs, accum, src_v, dst_v, w_v, rows0, rows1, sg0, sg1, ss0, ss1):
        cid = lax.axis_index("c")
        sid = lax.axis_index("s")

        base_row = sid * RA
        pltpu.sync_copy(x2_hbm.at[cid].at[pl.ds(base_row, RA)],
                        xs.at[pl.ds(base_row, RA)])

        def zero_row(r, _):
            z = jnp.zeros((L,), jnp.float32)
            for k in range(DH // L):
                rows0[r, pl.ds(k * L, L)] = z
            return 0
        lax.fori_loop(0, CHUNK, zero_row, 0)
        full = RA // CHUNK
        rem = RA - full * CHUNK
        for j in range(full):
            pltpu.sync_copy(rows0, accum.at[pl.ds(base_row + j * CHUNK, CHUNK)])
        pltpu.sync_copy(rows0.at[pl.ds(0, rem)],
                        accum.at[pl.ds(base_row + full * CHUNK, rem)])

        @pl.when(sid == NS - 1)
        def _():
            pltpu.sync_copy(x2_hbm.at[cid].at[pl.ds(NS * RA, TAIL)],
                            xs.at[pl.ds(NS * RA, TAIL)])
            pltpu.sync_copy(rows0.at[pl.ds(0, TAIL)],
                            accum.at[pl.ds(NS * RA, TAIL)])
        plsc.subcore_barrier()

        cbase = sid * n_chunks
        qr = n_chunks // NQ
        npairs = qr // 2

        def scale(buf, j):
            def scale_group(g, _):
                wvec = w_v[j, pl.ds(g * L, L)]
                for t in range(L):
                    e = g * L + t
                    w = wvec[t]
                    for k in range(DH // L):
                        buf[e, pl.ds(k * L, L)] = buf[e, pl.ds(k * L, L)] * w
                return 0
            lax.fori_loop(0, CHUNK // L, scale_group, 0)

        def pair_body(jj, _):
            j0 = 2 * jj
            j1 = j0 + 1
            pltpu.make_async_copy(xs.at[src_v.at[j0]], rows0, sg0).wait()
            dg1 = pltpu.async_copy(xs.at[src_v.at[j1]], rows1, sg1)
            scale(rows0, j0)
            ds0 = pltpu.async_copy(rows0, accum.at[dst_v.at[j0]], ss0, add=True)
            dg1.wait()
            ds0.wait()

            @pl.when(jj < npairs - 1)
            def _():
                pltpu.async_copy(xs.at[src_v.at[j0 + 2]], rows0, sg0)

            scale(rows1, j1)
            pltpu.async_copy(rows1, accum.at[dst_v.at[j1]], ss1, add=True).wait()
            return 0

        for q in range(NQ):
            pltpu.sync_copy(src_hbm.at[pl.ds(cbase + q * qr, qr)], src_v)
            pltpu.sync_copy(dst_hbm.at[pl.ds(cbase + q * qr, qr)], dst_v)
            pltpu.sync_copy(w_hbm.at[pl.ds(cbase + q * qr, qr)], w_v)
            pltpu.async_copy(xs.at[src_v.at[0]], rows0, sg0)
            lax.fori_loop(0, npairs, pair_body, 0)
        plsc.subcore_barrier()

        pltpu.sync_copy(accum.at[pl.ds(base_row, RA)],
                        out_hbm.at[cid].at[pl.ds(base_row, RA)])

        @pl.when(sid == NS - 1)
        def _():
            pltpu.sync_copy(accum.at[pl.ds(NS * RA, TAIL)],
                            out_hbm.at[cid].at[pl.ds(NS * RA, TAIL)])

    return agg(x2, src2d, dst2d, w2d)


def _ln_kernel(h0_ref, h1_ref, x_ref, g_ref, b_ref, o_ref):
    h = jnp.concatenate([h0_ref[...], h1_ref[...]], axis=1)
    y = h + x_ref[...]
    mean = jnp.mean(y, axis=1, keepdims=True)
    c = y - mean
    var = jnp.mean(c * c, axis=1, keepdims=True)
    o_ref[...] = c * lax.rsqrt(var + 1e-5) * g_ref[...] + b_ref[...]


def _ln(h0, h1, x, gamma, beta):
    blk = 2000
    grid = N // blk
    return pl.pallas_call(
        _ln_kernel,
        grid=(grid,),
        in_specs=[
            pl.BlockSpec((blk, DH), lambda i: (i, 0)),
            pl.BlockSpec((blk, DH), lambda i: (i, 0)),
            pl.BlockSpec((blk, D), lambda i: (i, 0)),
            pl.BlockSpec((1, D), lambda i: (0, 0)),
            pl.BlockSpec((1, D), lambda i: (0, 0)),
        ],
        out_specs=pl.BlockSpec((blk, D), lambda i: (i, 0)),
        out_shape=jax.ShapeDtypeStruct((N, D), jnp.float32),
    )(h0, h1, x, gamma.reshape(1, D), beta.reshape(1, D))


def kernel(x, edge_index, edge_weight, gamma, beta):
    e = edge_weight.shape[0]
    n_chunks = -(-e // (NS * CHUNK))
    align = NQ * 8
    n_chunks = -(-n_chunks // align) * align
    ep = NS * n_chunks * CHUNK
    pad = ep - e
    dst = jnp.pad(edge_index[0], (0, pad)).reshape(NS * n_chunks, CHUNK)
    src = jnp.pad(edge_index[1], (0, pad)).reshape(NS * n_chunks, CHUNK)
    w = jnp.pad(edge_weight, (0, pad)).reshape(NS * n_chunks, CHUNK)
    x2 = jnp.transpose(x.reshape(N, NC, DH), (1, 0, 2))
    parts = _sc_aggregate(x2, src, dst, w, n_chunks)
    return _ln(parts[0], parts[1], x, gamma, beta)


# speedup vs baseline: 1.0139x; 1.0139x over previous
import functools

import jax
import jax.numpy as jnp
from jax import lax
from jax.experimental import pallas as pl
from jax.experimental.pallas import tpu as pltpu
from jax.experimental.pallas import tpu_sc as plsc

NC = 2
NS = 16
L = 16

N = 10000
D = 128
DH = D // NC
CHUNK = 128
NQ = 4
RA = 624
TAIL = N - NS * RA


def _sc_aggregate(x2, src2d, dst2d, w2d, n_chunks):
    mesh = plsc.VectorSubcoreMesh(core_axis_name="c", subcore_axis_name="s")

    @functools.partial(
        pl.kernel,
        out_type=jax.ShapeDtypeStruct((NC, N, DH), jnp.float32),
        mesh=mesh,
        compiler_params=pltpu.CompilerParams(use_tc_tiling_on_sc=False),
        scratch_types=[
            pltpu.VMEM_SHARED((N, DH), jnp.float32),
            pltpu.VMEM_SHARED((N, DH), jnp.float32),
            pltpu.VMEM((2, n_chunks // NQ, CHUNK), jnp.int32),
            pltpu.VMEM((2, n_chunks // NQ, CHUNK), jnp.int32),
            pltpu.VMEM((2, n_chunks // NQ, CHUNK), jnp.float32),
            pltpu.VMEM((CHUNK, DH), jnp.float32),
            pltpu.VMEM((CHUNK, DH), jnp.float32),
            pltpu.SemaphoreType.DMA,
            pltpu.SemaphoreType.DMA,
            pltpu.SemaphoreType.DMA,
            pltpu.SemaphoreType.DMA,
            pltpu.SemaphoreType.DMA,
        ],
    )
    def agg(x2_hbm, src_hbm, dst_hbm, w_hbm, out_hbm,
            xs, accum, src_2v, dst_2v, w_2v, rows0, rows1, sg0, sg1, ss0, ss1, si):
        cid = lax.axis_index("c")
        sid = lax.axis_index("s")

        base_row = sid * RA
        pltpu.sync_copy(x2_hbm.at[cid].at[pl.ds(base_row, RA)],
                        xs.at[pl.ds(base_row, RA)])

        def zero_row(r, _):
            z = jnp.zeros((L,), jnp.float32)
            for k in range(DH // L):
                rows0[r, pl.ds(k * L, L)] = z
            return 0
        lax.fori_loop(0, CHUNK, zero_row, 0)
        full = RA // CHUNK
        rem = RA - full * CHUNK
        for j in range(full):
            pltpu.sync_copy(rows0, accum.at[pl.ds(base_row + j * CHUNK, CHUNK)])
        pltpu.sync_copy(rows0.at[pl.ds(0, rem)],
                        accum.at[pl.ds(base_row + full * CHUNK, rem)])

        @pl.when(sid == NS - 1)
        def _():
            pltpu.sync_copy(x2_hbm.at[cid].at[pl.ds(NS * RA, TAIL)],
                            xs.at[pl.ds(NS * RA, TAIL)])
            pltpu.sync_copy(rows0.at[pl.ds(0, TAIL)],
                            accum.at[pl.ds(NS * RA, TAIL)])
        plsc.subcore_barrier()

        cbase = sid * n_chunks
        qr = n_chunks // NQ
        npairs = qr // 2

        def scale(buf, w_v, j):
            def scale_group(g, _):
                wvec = w_v[j, pl.ds(g * L, L)]
                for t in range(L):
                    e = g * L + t
                    w = wvec[t]
                    for k in range(DH // L):
                        buf[e, pl.ds(k * L, L)] = buf[e, pl.ds(k * L, L)] * w
                return 0
            lax.fori_loop(0, CHUNK // L, scale_group, 0)

        def make_pair_body(src_v, dst_v, w_v):
            def pair_body(jj, _):
                j0 = 2 * jj
                j1 = j0 + 1

                @pl.when(jj > 0)
                def _():
                    pltpu.make_async_copy(rows1, accum.at[dst_v.at[0]], ss1).wait()

                pltpu.make_async_copy(xs.at[src_v.at[j0]], rows0, sg0).wait()
                dg1 = pltpu.async_copy(xs.at[src_v.at[j1]], rows1, sg1)
                scale(rows0, w_v, j0)
                ds0 = pltpu.async_copy(rows0, accum.at[dst_v.at[j0]], ss0, add=True)
                dg1.wait()
                ds0.wait()

                @pl.when(jj < npairs - 1)
                def _():
                    pltpu.async_copy(xs.at[src_v.at[j0 + 2]], rows0, sg0)

                scale(rows1, w_v, j1)
                pltpu.async_copy(rows1, accum.at[dst_v.at[j1]], ss1, add=True)
                return 0
            return pair_body

        for q in range(NQ):
            p = q % 2
            src_v, dst_v, w_v = src_2v.at[p], dst_2v.at[p], w_2v.at[p]
            if q == 0:
                pltpu.sync_copy(src_hbm.at[pl.ds(cbase, qr)], src_v)
                pltpu.sync_copy(dst_hbm.at[pl.ds(cbase, qr)], dst_v)
                pltpu.sync_copy(w_hbm.at[pl.ds(cbase, qr)], w_v)
            else:
                off = cbase + q * qr
                pltpu.make_async_copy(src_hbm.at[pl.ds(off, qr)], src_v, si).wait()
                pltpu.make_async_copy(dst_hbm.at[pl.ds(off, qr)], dst_v, si).wait()
                pltpu.make_async_copy(w_hbm.at[pl.ds(off, qr)], w_v, si).wait()
            if q + 1 < NQ:
                noff = cbase + (q + 1) * qr
                nset = (q + 1) % 2
                pltpu.async_copy(src_hbm.at[pl.ds(noff, qr)], src_2v.at[nset], si)
                pltpu.async_copy(dst_hbm.at[pl.ds(noff, qr)], dst_2v.at[nset], si)
                pltpu.async_copy(w_hbm.at[pl.ds(noff, qr)], w_2v.at[nset], si)
            pltpu.async_copy(xs.at[src_v.at[0]], rows0, sg0)
            lax.fori_loop(0, npairs, make_pair_body(src_v, dst_v, w_v), 0)
            pltpu.make_async_copy(rows1, accum.at[dst_v.at[0]], ss1).wait()
        plsc.subcore_barrier()

        pltpu.sync_copy(accum.at[pl.ds(base_row, RA)],
                        out_hbm.at[cid].at[pl.ds(base_row, RA)])

        @pl.when(sid == NS - 1)
        def _():
            pltpu.sync_copy(accum.at[pl.ds(NS * RA, TAIL)],
                            out_hbm.at[cid].at[pl.ds(NS * RA, TAIL)])

    return agg(x2, src2d, dst2d, w2d)


def _ln_kernel(h0_ref, h1_ref, x_ref, g_ref, b_ref, o_ref):
    h = jnp.concatenate([h0_ref[...], h1_ref[...]], axis=1)
    y = h + x_ref[...]
    mean = jnp.mean(y, axis=1, keepdims=True)
    c = y - mean
    var = jnp.mean(c * c, axis=1, keepdims=True)
    o_ref[...] = c * lax.rsqrt(var + 1e-5) * g_ref[...] + b_ref[...]


def _ln(h0, h1, x, gamma, beta):
    blk = 2000
    grid = N // blk
    return pl.pallas_call(
        _ln_kernel,
        grid=(grid,),
        in_specs=[
            pl.BlockSpec((blk, DH), lambda i: (i, 0)),
            pl.BlockSpec((blk, DH), lambda i: (i, 0)),
            pl.BlockSpec((blk, D), lambda i: (i, 0)),
            pl.BlockSpec((1, D), lambda i: (0, 0)),
            pl.BlockSpec((1, D), lambda i: (0, 0)),
        ],
        out_specs=pl.BlockSpec((blk, D), lambda i: (i, 0)),
        out_shape=jax.ShapeDtypeStruct((N, D), jnp.float32),
    )(h0, h1, x, gamma.reshape(1, D), beta.reshape(1, D))


def kernel(x, edge_index, edge_weight, gamma, beta):
    e = edge_weight.shape[0]
    n_chunks = -(-e // (NS * CHUNK))
    align = NQ * 8
    n_chunks = -(-n_chunks // align) * align
    ep = NS * n_chunks * CHUNK
    pad = ep - e
    dst = jnp.pad(edge_index[0], (0, pad)).reshape(NS * n_chunks, CHUNK)
    src = jnp.pad(edge_index[1], (0, pad)).reshape(NS * n_chunks, CHUNK)
    w = jnp.pad(edge_weight, (0, pad)).reshape(NS * n_chunks, CHUNK)
    x2 = jnp.transpose(x.reshape(N, NC, DH), (1, 0, 2))
    parts = _sc_aggregate(x2, src, dst, w, n_chunks)
    return _ln(parts[0], parts[1], x, gamma, beta)


# speedup vs baseline: 1.9789x; 1.9519x over previous
import functools

import jax
import jax.numpy as jnp
from jax import lax
from jax.experimental import pallas as pl
from jax.experimental.pallas import tpu as pltpu
from jax.experimental.pallas import tpu_sc as plsc

NC = 2
NS = 16
L = 16

N = 10000
D = 128
DH = D // NC
CHUNK = 128
NQ = 4
RA = 624
TAIL = N - NS * RA


def _sc_aggregate(x2, src2d, dst2d, w2d, n_chunks):
    mesh = plsc.VectorSubcoreMesh(core_axis_name="c", subcore_axis_name="s")

    @functools.partial(
        pl.kernel,
        out_type=jax.ShapeDtypeStruct((NC, N, DH), jnp.float32),
        mesh=mesh,
        compiler_params=pltpu.CompilerParams(use_tc_tiling_on_sc=False),
        scratch_types=[
            pltpu.VMEM_SHARED((N, DH), jnp.float32),
            pltpu.VMEM_SHARED((N, DH), jnp.float32),
            pltpu.VMEM((2, n_chunks // NQ, CHUNK), jnp.int32),
            pltpu.VMEM((2, n_chunks // NQ, CHUNK), jnp.int32),
            pltpu.VMEM((2, n_chunks // NQ, CHUNK), jnp.float32),
            pltpu.VMEM((CHUNK, DH), jnp.float32),
            pltpu.VMEM((CHUNK, DH), jnp.float32),
            pltpu.SemaphoreType.DMA,
            pltpu.SemaphoreType.DMA,
            pltpu.SemaphoreType.DMA,
            pltpu.SemaphoreType.DMA,
            pltpu.SemaphoreType.DMA,
        ],
    )
    def agg(x2_hbm, src_hbm, dst_hbm, w_hbm, out_hbm,
            xs, accum, src_2v, dst_2v, w_2v, rows0, rows1, sg0, sg1, ss0, ss1, si):
        cid = lax.axis_index("c")
        sid = lax.axis_index("s")

        base_row = sid * RA
        pltpu.sync_copy(x2_hbm.at[cid].at[pl.ds(base_row, RA)],
                        xs.at[pl.ds(base_row, RA)])

        def zero_row(r, _):
            z = jnp.zeros((L,), jnp.float32)
            for k in range(DH // L):
                rows0[r, pl.ds(k * L, L)] = z
            return 0
        lax.fori_loop(0, CHUNK, zero_row, 0)
        full = RA // CHUNK
        rem = RA - full * CHUNK
        for j in range(full):
            pltpu.sync_copy(rows0, accum.at[pl.ds(base_row + j * CHUNK, CHUNK)])
        pltpu.sync_copy(rows0.at[pl.ds(0, rem)],
                        accum.at[pl.ds(base_row + full * CHUNK, rem)])

        @pl.when(sid == NS - 1)
        def _():
            pltpu.sync_copy(x2_hbm.at[cid].at[pl.ds(NS * RA, TAIL)],
                            xs.at[pl.ds(NS * RA, TAIL)])
            pltpu.sync_copy(rows0.at[pl.ds(0, TAIL)],
                            accum.at[pl.ds(NS * RA, TAIL)])
        plsc.subcore_barrier()

        cbase = sid * n_chunks
        qr = n_chunks // NQ
        npairs = qr // 2

        def scale(buf, w_v, j):
            def scale_group(g2, _):
                for u in range(2):
                    g = 2 * g2 + u
                    wvec = w_v[j, pl.ds(g * L, L)]
                    for t in range(L):
                        e = g * L + t
                        w = wvec[t]
                        for k in range(DH // L):
                            buf[e, pl.ds(k * L, L)] = buf[e, pl.ds(k * L, L)] * w
                return 0
            lax.fori_loop(0, CHUNK // L // 2, scale_group, 0)

        def make_pair_body(src_v, dst_v, w_v):
            def pair_body(jj, _):
                j0 = 2 * jj
                j1 = j0 + 1

                @pl.when(jj > 0)
                def _():
                    pltpu.make_async_copy(rows1, accum.at[dst_v.at[0]], ss1).wait()

                pltpu.make_async_copy(xs.at[src_v.at[j0]], rows0, sg0).wait()
                dg1 = pltpu.async_copy(xs.at[src_v.at[j1]], rows1, sg1)
                scale(rows0, w_v, j0)
                ds0 = pltpu.async_copy(rows0, accum.at[dst_v.at[j0]], ss0, add=True)
                dg1.wait()
                ds0.wait()

                @pl.when(jj < npairs - 1)
                def _():
                    pltpu.async_copy(xs.at[src_v.at[j0 + 2]], rows0, sg0)

                scale(rows1, w_v, j1)
                pltpu.async_copy(rows1, accum.at[dst_v.at[j1]], ss1, add=True)
                return 0
            return pair_body

        for q in range(NQ):
            p = q % 2
            src_v, dst_v, w_v = src_2v.at[p], dst_2v.at[p], w_2v.at[p]
            if q == 0:
                pltpu.sync_copy(src_hbm.at[pl.ds(cbase, qr)], src_v)
                pltpu.sync_copy(dst_hbm.at[pl.ds(cbase, qr)], dst_v)
                pltpu.sync_copy(w_hbm.at[pl.ds(cbase, qr)], w_v)
            else:
                off = cbase + q * qr
                pltpu.make_async_copy(src_hbm.at[pl.ds(off, qr)], src_v, si).wait()
                pltpu.make_async_copy(dst_hbm.at[pl.ds(off, qr)], dst_v, si).wait()
                pltpu.make_async_copy(w_hbm.at[pl.ds(off, qr)], w_v, si).wait()
            if q + 1 < NQ:
                noff = cbase + (q + 1) * qr
                nset = (q + 1) % 2
                pltpu.async_copy(src_hbm.at[pl.ds(noff, qr)], src_2v.at[nset], si)
                pltpu.async_copy(dst_hbm.at[pl.ds(noff, qr)], dst_2v.at[nset], si)
                pltpu.async_copy(w_hbm.at[pl.ds(noff, qr)], w_2v.at[nset], si)
            pltpu.async_copy(xs.at[src_v.at[0]], rows0, sg0)
            lax.fori_loop(0, npairs, make_pair_body(src_v, dst_v, w_v), 0)
            pltpu.make_async_copy(rows1, accum.at[dst_v.at[0]], ss1).wait()
        plsc.subcore_barrier()

        pltpu.sync_copy(accum.at[pl.ds(base_row, RA)],
                        out_hbm.at[cid].at[pl.ds(base_row, RA)])

        @pl.when(sid == NS - 1)
        def _():
            pltpu.sync_copy(accum.at[pl.ds(NS * RA, TAIL)],
                            out_hbm.at[cid].at[pl.ds(NS * RA, TAIL)])

    return agg(x2, src2d, dst2d, w2d)


def _ln_kernel(h0_ref, h1_ref, x_ref, g_ref, b_ref, o_ref):
    h = jnp.concatenate([h0_ref[...], h1_ref[...]], axis=1)
    y = h + x_ref[...]
    mean = jnp.mean(y, axis=1, keepdims=True)
    c = y - mean
    var = jnp.mean(c * c, axis=1, keepdims=True)
    o_ref[...] = c * lax.rsqrt(var + 1e-5) * g_ref[...] + b_ref[...]


def _ln(h0, h1, x, gamma, beta):
    blk = 2000
    grid = N // blk
    return pl.pallas_call(
        _ln_kernel,
        grid=(grid,),
        in_specs=[
            pl.BlockSpec((blk, DH), lambda i: (i, 0)),
            pl.BlockSpec((blk, DH), lambda i: (i, 0)),
            pl.BlockSpec((blk, D), lambda i: (i, 0)),
            pl.BlockSpec((1, D), lambda i: (0, 0)),
            pl.BlockSpec((1, D), lambda i: (0, 0)),
        ],
        out_specs=pl.BlockSpec((blk, D), lambda i: (i, 0)),
        out_shape=jax.ShapeDtypeStruct((N, D), jnp.float32),
    )(h0, h1, x, gamma.reshape(1, D), beta.reshape(1, D))


def kernel(x, edge_index, edge_weight, gamma, beta):
    e = edge_weight.shape[0]
    n_chunks = -(-e // (NS * CHUNK))
    align = NQ * 8
    n_chunks = -(-n_chunks // align) * align
    ep = NS * n_chunks * CHUNK
    pad = ep - e
    dst = jnp.pad(edge_index[0], (0, pad)).reshape(NS * n_chunks, CHUNK)
    src = jnp.pad(edge_index[1], (0, pad)).reshape(NS * n_chunks, CHUNK)
    w = jnp.pad(edge_weight, (0, pad)).reshape(NS * n_chunks, CHUNK)
    x2 = jnp.transpose(x.reshape(N, NC, DH), (1, 0, 2))
    parts = _sc_aggregate(x2, src, dst, w, n_chunks)
    return _ln(parts[0], parts[1], x, gamma, beta)
